# baseline (speedup 1.0000x reference)
import functools

import jax
import jax.numpy as jnp
from jax.experimental import pallas as pl

_K = 16


def _knn_block_kernel(q_ref, k_ref, idx_ref, dist_ref, *, n_keys):
    q = q_ref[0]
    keys = k_ref[0]

    gram = jax.lax.dot_general(
        keys, q, (((0,), (0,)), ((), ())),
        preferred_element_type=jnp.float32)

    qn = jnp.sum(q * q, axis=0)
    kn = jnp.sum(keys * keys, axis=0)
    d2 = (kn[:, None] + qn[None, :]) - 2.0 * gram
    vals = jnp.maximum(d2, 0.0)

    r = vals.shape[1]
    iota = jax.lax.broadcasted_iota(jnp.int32, (n_keys, r), 0)
    for kk in range(_K):
        mv = jnp.min(vals, axis=0, keepdims=True)
        idx = jnp.min(jnp.where(vals == mv, iota, n_keys), axis=0,
                      keepdims=True)
        dist_ref[0, kk, :] = mv[0, :]
        idx_ref[0, kk, :] = idx[0, :]
        vals = jnp.where(iota == idx, jnp.inf, vals)


def kernel(positions):
    b, d, n = positions.shape
    r = 1024
    grid = (b, n // r)
    fn = functools.partial(_knn_block_kernel, n_keys=n)
    idx, dist = pl.pallas_call(
        fn,
        grid=grid,
        in_specs=[
            pl.BlockSpec((1, d, r), lambda bi, ri: (bi, 0, ri)),
            pl.BlockSpec((1, d, n), lambda bi, ri: (bi, 0, 0)),
        ],
        out_specs=[
            pl.BlockSpec((1, _K, r), lambda bi, ri: (bi, 0, ri)),
            pl.BlockSpec((1, _K, r), lambda bi, ri: (bi, 0, ri)),
        ],
        out_shape=[
            jax.ShapeDtypeStruct((b, _K, n), jnp.int32),
            jax.ShapeDtypeStruct((b, _K, n), jnp.float32),
        ],
    )(positions, positions)
    return idx, dist


# speedup vs baseline: 1.0003x; 1.0003x over previous
import functools

import jax
import jax.numpy as jnp
from jax.experimental import pallas as pl
from jax.experimental.pallas import tpu as pltpu

_K = 16


def _knn_block_kernel(q_ref, k_ref, idx_ref, dist_ref, *, n_keys):
    q = q_ref[0]
    keys = k_ref[0]

    gram = jax.lax.dot_general(
        keys, q, (((0,), (0,)), ((), ())),
        preferred_element_type=jnp.float32)

    qn = jnp.sum(q * q, axis=0)
    kn = jnp.sum(keys * keys, axis=0)
    d2 = (kn[:, None] + qn[None, :]) - 2.0 * gram
    vals = jnp.maximum(d2, 0.0)

    r = vals.shape[1]
    iota = jax.lax.broadcasted_iota(jnp.int32, (n_keys, r), 0)
    for kk in range(_K):
        mv = jnp.min(vals, axis=0, keepdims=True)
        idx = jnp.min(jnp.where(vals == mv, iota, n_keys), axis=0,
                      keepdims=True)
        dist_ref[0, kk, :] = mv[0, :]
        idx_ref[0, kk, :] = idx[0, :]
        vals = jnp.where(iota == idx, jnp.inf, vals)


def kernel(positions):
    b, d, n = positions.shape
    r = 1024
    grid = (b, n // r)
    fn = functools.partial(_knn_block_kernel, n_keys=n)
    idx, dist = pl.pallas_call(
        fn,
        grid=grid,
        in_specs=[
            pl.BlockSpec((1, d, r), lambda bi, ri: (bi, 0, ri)),
            pl.BlockSpec((1, d, n), lambda bi, ri: (bi, 0, 0)),
        ],
        out_specs=[
            pl.BlockSpec((1, _K, r), lambda bi, ri: (bi, 0, ri)),
            pl.BlockSpec((1, _K, r), lambda bi, ri: (bi, 0, ri)),
        ],
        out_shape=[
            jax.ShapeDtypeStruct((b, _K, n), jnp.int32),
            jax.ShapeDtypeStruct((b, _K, n), jnp.float32),
        ],
        compiler_params=pltpu.CompilerParams(
            dimension_semantics=("parallel", "parallel")),
    )(positions, positions)
    return idx, dist


# speedup vs baseline: 1.1455x; 1.1451x over previous
import functools

import jax
import jax.numpy as jnp
from jax.experimental import pallas as pl
from jax.experimental.pallas import tpu as pltpu

_K = 16


def _knn_block_kernel(q_ref, k_ref, idx_ref, dist_ref, *, n_keys):
    q = q_ref[0]
    keys = k_ref[0]

    gram = jax.lax.dot_general(
        keys, q, (((0,), (0,)), ((), ())),
        preferred_element_type=jnp.float32)

    qn = jnp.sum(q * q, axis=0)
    kn = jnp.sum(keys * keys, axis=0)
    d2 = (kn[:, None] + qn[None, :]) - 2.0 * gram
    vals = jnp.maximum(d2, 0.0)

    r = vals.shape[1]
    half = n_keys // 2
    av = vals[:half, :]
    bv = vals[half:, :]
    ih = jax.lax.broadcasted_iota(jnp.int32, (half, r), 0)
    le = av <= bv
    pm = jnp.where(le, av, bv)
    pidx = jnp.where(le, ih, ih + half)
    sv = jnp.where(le, bv, av)
    sidx = jnp.where(le, ih + half, ih)
    for kk in range(_K):
        mv = jnp.min(pm, axis=0, keepdims=True)
        idx = jnp.min(jnp.where(pm == mv, pidx, n_keys), axis=0,
                      keepdims=True)
        dist_ref[0, kk, :] = mv[0, :]
        idx_ref[0, kk, :] = idx[0, :]
        wm = pidx == idx
        newv = jnp.where(pidx == sidx, jnp.inf, sv)
        pm = jnp.where(wm, newv, pm)
        pidx = jnp.where(wm, sidx, pidx)


def kernel(positions):
    b, d, n = positions.shape
    r = 1024
    grid = (b, n // r)
    fn = functools.partial(_knn_block_kernel, n_keys=n)
    idx, dist = pl.pallas_call(
        fn,
        grid=grid,
        in_specs=[
            pl.BlockSpec((1, d, r), lambda bi, ri: (bi, 0, ri)),
            pl.BlockSpec((1, d, n), lambda bi, ri: (bi, 0, 0)),
        ],
        out_specs=[
            pl.BlockSpec((1, _K, r), lambda bi, ri: (bi, 0, ri)),
            pl.BlockSpec((1, _K, r), lambda bi, ri: (bi, 0, ri)),
        ],
        out_shape=[
            jax.ShapeDtypeStruct((b, _K, n), jnp.int32),
            jax.ShapeDtypeStruct((b, _K, n), jnp.float32),
        ],
        compiler_params=pltpu.CompilerParams(
            dimension_semantics=("parallel", "parallel")),
    )(positions, positions)
    return idx, dist
